# baseline (speedup 1.0000x reference)
import functools

import jax
import jax.numpy as jnp
from jax import lax
from jax.experimental import pallas as pl
from jax.experimental.pallas import tpu as pltpu
from jax.experimental.pallas import tpu_sc as plsc

B = 16384
L = 50
D = 32
V = 1000000
LANES = 16

NC, NS = 2, 16
NW = NC * NS
RPB = 2
IPB = RPB * L
NBLK = B // RPB
BPW = NBLK // NW
ROWS_PW = B // NW
RING = 4

_mesh = plsc.VectorSubcoreMesh(
    core_axis_name="c", subcore_axis_name="s", num_cores=NC, num_subcores=NS
)

_GATHER_DNUMS = lax.GatherDimensionNumbers(
    offset_dims=(), collapsed_slice_dims=(0,), start_index_map=(0,)
)


def _bcast_lane(vec, lane):
    idx = jnp.full((LANES, 1), lane, dtype=jnp.int32)
    return lax.gather(
        vec, idx, _GATHER_DNUMS, (1,),
        mode=lax.GatherScatterMode.PROMISE_IN_BOUNDS,
    )


@functools.partial(
    pl.kernel,
    out_type=jax.ShapeDtypeStruct((B, D), jnp.float32),
    mesh=_mesh,
    scratch_types=[
        pltpu.VMEM((BPW, IPB), jnp.int32),
        pltpu.VMEM((BPW, IPB), jnp.int32),
        pltpu.VMEM((RING, IPB, 128), jnp.float32),
        pltpu.VMEM((ROWS_PW, D), jnp.float32),
        pltpu.SemaphoreType.DMA,
    ]
    + [pltpu.SemaphoreType.DMA] * RING,
    compiler_params=pltpu.CompilerParams(
        use_tc_tiling_on_sc=False, needs_layout_passes=False
    ),
)
def _sc_sum(idx_hbm, tlin_hbm, out_hbm, idx_v, idxq_v, rows_v, out_v,
            sem_i, *sems):
    wid = lax.axis_index("s") * NC + lax.axis_index("c")
    blk0 = wid * BPW

    idx_cp = pltpu.make_async_copy(
        idx_hbm.at[pl.ds(blk0, BPW), :], idx_v, sem_i
    )
    idx_cp.start()
    idx_cp.wait()

    def shift_row(r, carry):
        for off in (0, 16, 32, 48, 64, 80, IPB - LANES):
            idxq_v[r, pl.ds(off, LANES)] = (
                idx_v[r, pl.ds(off, LANES)] >> 2
            )
        return carry

    lax.fori_loop(0, BPW, shift_row, 0)

    for s in range(RING):
        pltpu.make_async_copy(
            tlin_hbm.at[idxq_v.at[s]], rows_v.at[s], sems[s]
        ).start()

    iota = lax.iota(jnp.int32, LANES)

    def body(k, carry):
        for s in range(RING):
            j = k * RING + s
            pltpu.make_async_copy(
                tlin_hbm.at[idxq_v.at[j]], rows_v.at[s], sems[s]
            ).wait()
            for r in range(RPB):
                base = r * L
                rem = []
                for g, off in enumerate((0, 16, 32, L - LANES)):
                    rem.append(idx_v[j, pl.ds(base + off, LANES)] & 3)
                a0 = jnp.zeros((LANES,), jnp.float32)
                a1 = jnp.zeros((LANES,), jnp.float32)
                for q in range(L):
                    g, lane = divmod(q, LANES)
                    if g >= 3:
                        g, lane = 3, q - (L - LANES)
                    col = _bcast_lane(rem[g], lane) * D + iota
                    srow = jnp.full((LANES,), s, jnp.int32)
                    grow = jnp.full((LANES,), base + q, jnp.int32)
                    a0 = a0 + plsc.load_gather(rows_v, [srow, grow, col])
                    a1 = a1 + plsc.load_gather(
                        rows_v, [srow, grow, col + LANES]
                    )
                orow = j * RPB + r
                out_v[orow, pl.ds(0, LANES)] = a0
                out_v[orow, pl.ds(LANES, LANES)] = a1
            nxt = j + RING

            @pl.when(nxt < BPW)
            def _():
                pltpu.make_async_copy(
                    tlin_hbm.at[idxq_v.at[nxt]], rows_v.at[s], sems[s]
                ).start()

        return carry

    lax.fori_loop(0, BPW // RING, body, 0)

    pltpu.sync_copy(out_v, out_hbm.at[pl.ds(wid * ROWS_PW, ROWS_PW), :])


def _fin_body(idx_ref, sums_ref, t0_ref, out_ref):
    idx = idx_ref[...]
    sums = sums_ref[...]
    t0 = t0_ref[...]
    cnt = jnp.sum((idx != 0).astype(jnp.float32), axis=1, keepdims=True)
    nz = jnp.float32(L) - cnt
    mean = (sums - nz * t0) / cnt
    nrm = jnp.sqrt(jnp.sum(mean * mean, axis=1, keepdims=True))
    out_ref[...] = mean / jnp.maximum(nrm, 1e-12)


_FIN_BLK = 1024

_fin = pl.pallas_call(
    _fin_body,
    grid=(B // _FIN_BLK,),
    in_specs=[
        pl.BlockSpec((_FIN_BLK, L), lambda i: (i, 0)),
        pl.BlockSpec((_FIN_BLK, D), lambda i: (i, 0)),
        pl.BlockSpec((1, D), lambda i: (0, 0)),
    ],
    out_specs=pl.BlockSpec((_FIN_BLK, D), lambda i: (i, 0)),
    out_shape=jax.ShapeDtypeStruct((B, D), jnp.float32),
)


@jax.jit
def kernel(ingredient_indices, table):
    idx2d = ingredient_indices.reshape(NBLK, IPB)
    tlin = table.reshape(V // 4, 128)
    sums = _sc_sum(idx2d, tlin)
    return _fin(ingredient_indices, sums, table[0:1])
